# baseline (speedup 1.0000x reference)
import functools
import jax
import jax.numpy as jnp
from jax import lax
from jax.experimental import pallas as pl
from jax.experimental.pallas import tpu as pltpu
from jax.experimental.pallas import tpu_sc as plsc

B = 4
N = 10000
NP = 10240
NR = NP // 128
E = 320000
EP = 327680
ER = EP // 128
F_IN = 128
D = 64

NC = 2
NS = 16
EPT = EP // NS
NBLK = EPT // 2048

_mesh = plsc.VectorSubcoreMesh(core_axis_name="c", subcore_axis_name="s")


NRED = NP // 10


def _zero_flat(ref, nwords):
    def body(i, _):
        ref[pl.ds(16 * i, 16)] = jnp.zeros((16,), jnp.float32)
        return 0
    lax.fori_loop(0, nwords // 16, body, 0)


def _slab_reduce_out(acc, slab, tmp, red, sid, out_slice):
    pltpu.sync_copy(acc, slab.at[sid])
    plsc.subcore_barrier()

    @pl.when(sid < 10)
    def _():
        pltpu.sync_copy(slab.at[0, pl.ds(NRED * sid, NRED)], red)

        def addk(k, _):
            pltpu.sync_copy(slab.at[k, pl.ds(NRED * sid, NRED)], tmp)

            def vadd(i, _):
                red[pl.ds(16 * i, 16)] = (red[pl.ds(16 * i, 16)]
                                          + tmp[pl.ds(16 * i, 16)])
                return 0
            lax.fori_loop(0, NRED // 16, vadd, 0)
            return 0
        lax.fori_loop(1, NS, addk, 0)
        out_slice(sid, red)
    plsc.subcore_barrier()


@functools.partial(
    pl.kernel,
    out_type=jax.ShapeDtypeStruct((B, 1, NP), jnp.float32),
    mesh=_mesh,
    compiler_params=pltpu.CompilerParams(needs_layout_passes=False, use_tc_tiling_on_sc=False),
    scratch_types=[
        pltpu.VMEM((16, 128), jnp.int32),
        pltpu.VMEM((16, 128), jnp.float32),
        pltpu.VMEM((NP,), jnp.float32),
        pltpu.VMEM((NRED,), jnp.float32),
        pltpu.VMEM((NRED,), jnp.float32),
        pltpu.VMEM_SHARED((NS, NP), jnp.float32),
    ],
)
def _deg_kernel(dst_hbm, ew_hbm, deg_hbm, dbuf, wbuf, acc, tmp, red, slab):
    cid = lax.axis_index("c")
    sid = lax.axis_index("s")
    row0 = sid * (EPT // 128)

    def per_graph(gi, _):
        b = 2 * cid + gi
        _zero_flat(acc, NP)

        def per_block(k, _):
            pltpu.sync_copy(dst_hbm.at[b, pl.ds(row0 + 16 * k, 16)], dbuf)
            pltpu.sync_copy(ew_hbm.at[b, pl.ds(row0 + 16 * k, 16)], wbuf)

            @plsc.parallel_loop(0, 16, step=1, unroll=4)
            def _(j):
                for u in range(8):
                    dvec = dbuf[j, pl.ds(16 * u, 16)]
                    wvec = wbuf[j, pl.ds(16 * u, 16)]
                    plsc.addupdate_scatter(acc, [dvec], wvec)
            return 0
        lax.fori_loop(0, NBLK, per_block, 0)

        def out_slice(t, red_ref):
            pltpu.sync_copy(red_ref, deg_hbm.at[b, 0, pl.ds(NRED * t, NRED)])
        _slab_reduce_out(acc, slab, tmp, red, sid, out_slice)
        return 0
    lax.fori_loop(0, B // NC, per_graph, 0)


EB = EP // 2048


@functools.partial(
    pl.kernel,
    out_type=jax.ShapeDtypeStruct((B, D, NP), jnp.float32),
    mesh=_mesh,
    compiler_params=pltpu.CompilerParams(needs_layout_passes=False, use_tc_tiling_on_sc=False),
    scratch_types=[
        pltpu.VMEM((2 * 2048,), jnp.int32),
        pltpu.VMEM((2 * 2048,), jnp.int32),
        pltpu.VMEM((2 * 2048,), jnp.float32),
        pltpu.VMEM((NP,), jnp.float32),
        pltpu.VMEM((NP,), jnp.float32),
        pltpu.VMEM((NP,), jnp.float32),
        pltpu.VMEM((NP,), jnp.float32),
        pltpu.VMEM((NP,), jnp.float32),
        pltpu.VMEM((NP,), jnp.float32),
        pltpu.VMEM((NP,), jnp.float32),
        pltpu.VMEM((NP,), jnp.float32),
        pltpu.SemaphoreType.DMA,
        pltpu.SemaphoreType.DMA,
    ],
)
def _spmm_kernel(g_hbm, src_hbm, dst_hbm, ew_hbm, s_hbm,
                 sbuf, dbuf, wbuf, gc0, gc1, gc2, gc3, ac0, ac1, ac2, ac3,
                 es0, es1):
    cid = lax.axis_index("c")
    sid = lax.axis_index("s")
    gcs = (gc0, gc1, gc2, gc3)
    acs = (ac0, ac1, ac2, ac3)
    c0 = 4 * sid

    def issue(b, k, slot, sem):
        for hbm, buf in ((src_hbm, sbuf), (dst_hbm, dbuf), (ew_hbm, wbuf)):
            pltpu.async_copy(hbm.at[b, pl.ds(2048 * k, 2048)],
                             buf.at[pl.ds(slot * 2048, 2048)], sem)

    def drain(b, slot, sem):
        for hbm, buf in ((src_hbm, sbuf), (dst_hbm, dbuf), (ew_hbm, wbuf)):
            pltpu.make_async_copy(hbm.at[b, pl.ds(0, 2048)],
                                  buf.at[pl.ds(slot * 2048, 2048)], sem).wait()

    def process(slot):
        @plsc.parallel_loop(0, 128, step=1, unroll=8)
        def _(j):
            off = slot * 2048 + 16 * j
            svec = sbuf[pl.ds(off, 16)]
            dvec = dbuf[pl.ds(off, 16)]
            wvec = wbuf[pl.ds(off, 16)]
            for c in range(4):
                vals = plsc.load_gather(gcs[c], [svec])
                plsc.addupdate_scatter(acs[c], [dvec], vals * wvec)

    def per_graph(gi, _):
        b = 2 * cid + gi
        for c in range(4):
            pltpu.sync_copy(g_hbm.at[b, c0 + c], gcs[c])
            _zero_flat(acs[c], NP)
        issue(b, 0, 0, es0)

        def pair(k2, _):
            k = 2 * k2
            issue(b, k + 1, 1, es1)
            drain(b, 0, es0)
            process(0)

            @pl.when(k2 < EB // 2 - 1)
            def _():
                issue(b, k + 2, 0, es0)
            drain(b, 1, es1)
            process(1)
            return 0
        lax.fori_loop(0, EB // 2, pair, 0)

        for c in range(4):
            pltpu.sync_copy(acs[c], s_hbm.at[b, c0 + c])
        return 0
    lax.fori_loop(0, B // NC, per_graph, 0)


@functools.partial(
    pl.kernel,
    out_type=jax.ShapeDtypeStruct((B, 1, NP), jnp.float32),
    mesh=_mesh,
    compiler_params=pltpu.CompilerParams(needs_layout_passes=False, use_tc_tiling_on_sc=False),
    scratch_types=[
        pltpu.VMEM((16, 128), jnp.int32),
        pltpu.VMEM((16, 128), jnp.int32),
        pltpu.VMEM((16, 128), jnp.float32),
        pltpu.VMEM((NP,), jnp.float32),
        pltpu.VMEM((NP,), jnp.float32),
        pltpu.VMEM((NRED,), jnp.float32),
        pltpu.VMEM((NRED,), jnp.float32),
        pltpu.VMEM_SHARED((NS, NP), jnp.float32),
    ],
)
def _spmv_kernel(g_hbm, src_hbm, dst_hbm, ew_hbm, s_hbm,
                 sbuf, dbuf, wbuf, gvec, acc, tmp, red, slab):
    cid = lax.axis_index("c")
    sid = lax.axis_index("s")
    row0 = sid * (EPT // 128)

    def per_graph(gi, _):
        b = 2 * cid + gi
        pltpu.sync_copy(g_hbm.at[b, 0], gvec)
        _zero_flat(acc, NP)

        def per_block(k, _):
            pltpu.sync_copy(src_hbm.at[b, pl.ds(row0 + 16 * k, 16)], sbuf)
            pltpu.sync_copy(dst_hbm.at[b, pl.ds(row0 + 16 * k, 16)], dbuf)
            pltpu.sync_copy(ew_hbm.at[b, pl.ds(row0 + 16 * k, 16)], wbuf)

            @plsc.parallel_loop(0, 16, step=1, unroll=4)
            def _(j):
                for u in range(8):
                    svec = sbuf[j, pl.ds(16 * u, 16)]
                    dvec = dbuf[j, pl.ds(16 * u, 16)]
                    wvec = wbuf[j, pl.ds(16 * u, 16)]
                    vals = plsc.load_gather(gvec, [svec])
                    plsc.addupdate_scatter(acc, [dvec], vals * wvec)
            return 0
        lax.fori_loop(0, NBLK, per_block, 0)

        def out_slice(t, red_ref):
            pltpu.sync_copy(red_ref, s_hbm.at[b, 0, pl.ds(NRED * t, NRED)])
        _slab_reduce_out(acc, slab, tmp, red, sid, out_slice)
        return 0
    lax.fori_loop(0, B // NC, per_graph, 0)


def _dis(deg):
    return jnp.where(deg > 0, lax.rsqrt(jnp.where(deg > 0, deg, 1.0)), 0.0)


def _k1t_body(deg_ref, xt_ref, wt_ref, o_ref):
    dis = _dis(deg_ref[0])
    o_ref[0] = jnp.dot(wt_ref[...], xt_ref[0],
                       preferred_element_type=jnp.float32) * dis


def _kmidt_body(deg_ref, st_ref, b_ref, wt_ref, o_ref):
    dis = _dis(deg_ref[0])
    x = jnp.maximum(st_ref[0] * dis + b_ref[...], 0.0)
    o_ref[0] = jnp.dot(wt_ref[...], x,
                       preferred_element_type=jnp.float32) * dis


def _kfinal_body(deg_ref, s_ref, b_ref, o_ref):
    deg = deg_ref[0]
    o = s_ref[0] * _dis(deg) + b_ref[...]
    idx = (lax.broadcasted_iota(jnp.int32, (NR, 128), 0) * 128
           + lax.broadcasted_iota(jnp.int32, (NR, 128), 1))
    mask = (idx >= 1) & (idx < N - 1)
    m = jnp.max(jnp.where(mask, o, -jnp.inf))
    ex = jnp.where(mask, jnp.exp(o - m), 0.0)
    o_ref[0] = ex / jnp.sum(ex)


_RB = 1280
_NRB = NP // _RB


def _tc_matmul1t(deg_r, xt, WT):
    return pl.pallas_call(
        _k1t_body,
        grid=(B, _NRB),
        in_specs=[
            pl.BlockSpec((1, 1, _RB), lambda b, i: (b, 0, i)),
            pl.BlockSpec((1, F_IN, _RB), lambda b, i: (b, 0, i)),
            pl.BlockSpec((D, F_IN), lambda b, i: (0, 0)),
        ],
        out_specs=pl.BlockSpec((1, D, _RB), lambda b, i: (b, 0, i)),
        out_shape=jax.ShapeDtypeStruct((B, D, NP), jnp.float32),
    )(deg_r, xt, WT)


def _tc_matmul_midt(deg_r, st, bias_c, WT):
    wo = WT.shape[0]
    return pl.pallas_call(
        _kmidt_body,
        grid=(B, _NRB),
        in_specs=[
            pl.BlockSpec((1, 1, _RB), lambda b, i: (b, 0, i)),
            pl.BlockSpec((1, D, _RB), lambda b, i: (b, 0, i)),
            pl.BlockSpec((D, 1), lambda b, i: (0, 0)),
            pl.BlockSpec((wo, D), lambda b, i: (0, 0)),
        ],
        out_specs=pl.BlockSpec((1, wo, _RB), lambda b, i: (b, 0, i)),
        out_shape=jax.ShapeDtypeStruct((B, wo, NP), jnp.float32),
    )(deg_r, st, bias_c, WT)


def _tc_final(deg_r, s4, b3):
    return pl.pallas_call(
        _kfinal_body,
        grid=(B,),
        in_specs=[
            pl.BlockSpec((1, NR, 128), lambda b: (b, 0, 0)),
            pl.BlockSpec((1, NR, 128), lambda b: (b, 0, 0)),
            pl.BlockSpec((1,), lambda b: (0,)),
        ],
        out_specs=pl.BlockSpec((1, NR, 128), lambda b: (b, 0, 0)),
        out_shape=jax.ShapeDtypeStruct((B, NR, 128), jnp.float32),
    )(deg_r, s4, b3)


def kernel(batch_feat, batch_edges, batch_attr, W1, b1, W2, b2, W22, b22, W3, b3):
    xp = jnp.pad(batch_feat, ((0, 0), (0, NP - N), (0, 0)))
    xt = jnp.swapaxes(xp, 1, 2)
    src = jnp.pad(batch_edges[:, 0, :], ((0, 0), (0, EP - E)))
    dst = jnp.pad(batch_edges[:, 1, :], ((0, 0), (0, EP - E)))
    ew = jnp.pad(batch_attr, ((0, 0), (0, EP - E)))
    src3 = src.reshape(B, ER, 128)
    dst3 = dst.reshape(B, ER, 128)
    ew3 = ew.reshape(B, ER, 128)

    deg_r = _deg_kernel(dst3, ew3)

    g1 = _tc_matmul1t(deg_r, xt, W1.T)
    s1 = _spmm_kernel(g1, src, dst, ew)
    g2 = _tc_matmul_midt(deg_r, s1, b1.reshape(D, 1), W2.T)
    s2 = _spmm_kernel(g2, src, dst, ew)
    g3 = _tc_matmul_midt(deg_r, s2, b2.reshape(D, 1), W22.T)
    s3 = _spmm_kernel(g3, src, dst, ew)

    g4 = _tc_matmul_midt(deg_r, s3, b22.reshape(D, 1), W3.T)
    s4 = _spmv_kernel(g4, src3, dst3, ew3)
    out = _tc_final(deg_r.reshape(B, NR, 128), s4.reshape(B, NR, 128), b3)
    return out.reshape(B, NP)[:, 1:N - 1]


# speedup vs baseline: 1.0819x; 1.0819x over previous
import functools
import jax
import jax.numpy as jnp
from jax import lax
from jax.experimental import pallas as pl
from jax.experimental.pallas import tpu as pltpu
from jax.experimental.pallas import tpu_sc as plsc

B = 4
N = 10000
NP = 10240
NR = NP // 128
E = 320000
EP = 327680
ER = EP // 128
F_IN = 128
D = 64

NC = 2
NS = 16
EPT = EP // NS
NBLK = EPT // 2048

_mesh = plsc.VectorSubcoreMesh(core_axis_name="c", subcore_axis_name="s")


NRED = NP // 10


def _zero_flat(ref, nwords):
    def body(i, _):
        ref[pl.ds(16 * i, 16)] = jnp.zeros((16,), jnp.float32)
        return 0
    lax.fori_loop(0, nwords // 16, body, 0)


def _slab_reduce_out(acc, slab, tmp, red, sid, out_slice):
    pltpu.sync_copy(acc, slab.at[sid])
    plsc.subcore_barrier()

    @pl.when(sid < 10)
    def _():
        pltpu.sync_copy(slab.at[0, pl.ds(NRED * sid, NRED)], red)

        def addk(k, _):
            pltpu.sync_copy(slab.at[k, pl.ds(NRED * sid, NRED)], tmp)

            def vadd(i, _):
                red[pl.ds(16 * i, 16)] = (red[pl.ds(16 * i, 16)]
                                          + tmp[pl.ds(16 * i, 16)])
                return 0
            lax.fori_loop(0, NRED // 16, vadd, 0)
            return 0
        lax.fori_loop(1, NS, addk, 0)
        out_slice(sid, red)
    plsc.subcore_barrier()


@functools.partial(
    pl.kernel,
    out_type=jax.ShapeDtypeStruct((B, 1, NP), jnp.float32),
    mesh=_mesh,
    compiler_params=pltpu.CompilerParams(needs_layout_passes=False, use_tc_tiling_on_sc=False),
    scratch_types=[
        pltpu.VMEM((EPT,), jnp.int32),
        pltpu.VMEM((EPT,), jnp.float32),
        pltpu.VMEM((NP,), jnp.float32),
        pltpu.VMEM((NRED,), jnp.float32),
        pltpu.VMEM((NRED,), jnp.float32),
        pltpu.VMEM_SHARED((NS, NP), jnp.float32),
    ],
)
def _deg_kernel(pk_hbm, ew_hbm, deg_hbm, pbuf, wbuf, acc, tmp, red, slab):
    cid = lax.axis_index("c")
    sid = lax.axis_index("s")
    e0 = sid * EPT

    def per_graph(gi, _):
        b = 2 * cid + gi
        _zero_flat(acc, NP)
        pltpu.sync_copy(pk_hbm.at[b, pl.ds(e0, EPT)], pbuf)
        pltpu.sync_copy(ew_hbm.at[b, pl.ds(e0, EPT)], wbuf)

        @plsc.parallel_loop(0, EPT // 16, step=1, unroll=8)
        def _(j):
            dvec = lax.shift_right_logical(pbuf[pl.ds(16 * j, 16)], 14)
            plsc.addupdate_scatter(acc, [dvec], wbuf[pl.ds(16 * j, 16)])

        def out_slice(t, red_ref):
            pltpu.sync_copy(red_ref, deg_hbm.at[b, 0, pl.ds(NRED * t, NRED)])
        _slab_reduce_out(acc, slab, tmp, red, sid, out_slice)
        return 0
    lax.fori_loop(0, B // NC, per_graph, 0)


EB = EP // 2048


@functools.partial(
    pl.kernel,
    out_type=jax.ShapeDtypeStruct((B, D, NP), jnp.float32),
    mesh=_mesh,
    compiler_params=pltpu.CompilerParams(needs_layout_passes=False, use_tc_tiling_on_sc=False),
    scratch_types=[
        pltpu.VMEM((2 * 2048,), jnp.int32),
        pltpu.VMEM((2 * 2048,), jnp.float32),
        pltpu.VMEM((NP,), jnp.float32),
        pltpu.VMEM((NP,), jnp.float32),
        pltpu.VMEM((NP,), jnp.float32),
        pltpu.VMEM((NP,), jnp.float32),
        pltpu.VMEM((NP,), jnp.float32),
        pltpu.VMEM((NP,), jnp.float32),
        pltpu.VMEM((NP,), jnp.float32),
        pltpu.VMEM((NP,), jnp.float32),
        pltpu.SemaphoreType.DMA,
        pltpu.SemaphoreType.DMA,
    ],
)
def _spmm_kernel(g_hbm, pk_hbm, ew_hbm, s_hbm,
                 pbuf, wbuf, gc0, gc1, gc2, gc3, ac0, ac1, ac2, ac3,
                 es0, es1):
    cid = lax.axis_index("c")
    sid = lax.axis_index("s")
    gcs = (gc0, gc1, gc2, gc3)
    acs = (ac0, ac1, ac2, ac3)
    c0 = 4 * sid

    def issue(b, k, slot, sem):
        for hbm, buf in ((pk_hbm, pbuf), (ew_hbm, wbuf)):
            pltpu.async_copy(hbm.at[b, pl.ds(2048 * k, 2048)],
                             buf.at[pl.ds(slot * 2048, 2048)], sem)

    def drain(b, slot, sem):
        for hbm, buf in ((pk_hbm, pbuf), (ew_hbm, wbuf)):
            pltpu.make_async_copy(hbm.at[b, pl.ds(0, 2048)],
                                  buf.at[pl.ds(slot * 2048, 2048)], sem).wait()

    def process(slot):
        @plsc.parallel_loop(0, 128, step=1, unroll=8)
        def _(j):
            off = slot * 2048 + 16 * j
            pvec = pbuf[pl.ds(off, 16)]
            svec = lax.bitwise_and(pvec, 16383)
            dvec = lax.shift_right_logical(pvec, 14)
            wvec = wbuf[pl.ds(off, 16)]
            for c in range(4):
                vals = plsc.load_gather(gcs[c], [svec])
                plsc.addupdate_scatter(acs[c], [dvec], vals * wvec)

    def per_graph(gi, _):
        b = 2 * cid + gi
        for c in range(4):
            pltpu.sync_copy(g_hbm.at[b, c0 + c], gcs[c])
            _zero_flat(acs[c], NP)
        issue(b, 0, 0, es0)

        def pair(k2, _):
            k = 2 * k2
            issue(b, k + 1, 1, es1)
            drain(b, 0, es0)
            process(0)

            @pl.when(k2 < EB // 2 - 1)
            def _():
                issue(b, k + 2, 0, es0)
            drain(b, 1, es1)
            process(1)
            return 0
        lax.fori_loop(0, EB // 2, pair, 0)

        for c in range(4):
            pltpu.sync_copy(acs[c], s_hbm.at[b, c0 + c])
        return 0
    lax.fori_loop(0, B // NC, per_graph, 0)


@functools.partial(
    pl.kernel,
    out_type=jax.ShapeDtypeStruct((B, 1, NP), jnp.float32),
    mesh=_mesh,
    compiler_params=pltpu.CompilerParams(needs_layout_passes=False, use_tc_tiling_on_sc=False),
    scratch_types=[
        pltpu.VMEM((EPT,), jnp.int32),
        pltpu.VMEM((EPT,), jnp.float32),
        pltpu.VMEM((NP,), jnp.float32),
        pltpu.VMEM((NP,), jnp.float32),
        pltpu.VMEM((NRED,), jnp.float32),
        pltpu.VMEM((NRED,), jnp.float32),
        pltpu.VMEM_SHARED((NS, NP), jnp.float32),
    ],
)
def _spmv_kernel(g_hbm, pk_hbm, ew_hbm, s_hbm,
                 pbuf, wbuf, gvec, acc, tmp, red, slab):
    cid = lax.axis_index("c")
    sid = lax.axis_index("s")
    e0 = sid * EPT

    def per_graph(gi, _):
        b = 2 * cid + gi
        pltpu.sync_copy(g_hbm.at[b, 0], gvec)
        _zero_flat(acc, NP)
        pltpu.sync_copy(pk_hbm.at[b, pl.ds(e0, EPT)], pbuf)
        pltpu.sync_copy(ew_hbm.at[b, pl.ds(e0, EPT)], wbuf)

        @plsc.parallel_loop(0, EPT // 16, step=1, unroll=8)
        def _(j):
            pvec = pbuf[pl.ds(16 * j, 16)]
            svec = lax.bitwise_and(pvec, 16383)
            dvec = lax.shift_right_logical(pvec, 14)
            vals = plsc.load_gather(gvec, [svec])
            plsc.addupdate_scatter(acc, [dvec], vals * wbuf[pl.ds(16 * j, 16)])

        def out_slice(t, red_ref):
            pltpu.sync_copy(red_ref, s_hbm.at[b, 0, pl.ds(NRED * t, NRED)])
        _slab_reduce_out(acc, slab, tmp, red, sid, out_slice)
        return 0
    lax.fori_loop(0, B // NC, per_graph, 0)


def _dis(deg):
    return jnp.where(deg > 0, lax.rsqrt(jnp.where(deg > 0, deg, 1.0)), 0.0)


def _k1t_body(deg_ref, xt_ref, wt_ref, o_ref):
    dis = _dis(deg_ref[0])
    o_ref[0] = jnp.dot(wt_ref[...], xt_ref[0],
                       preferred_element_type=jnp.float32) * dis


def _kmidt_body(deg_ref, st_ref, b_ref, wt_ref, o_ref):
    dis = _dis(deg_ref[0])
    x = jnp.maximum(st_ref[0] * dis + b_ref[...], 0.0)
    o_ref[0] = jnp.dot(wt_ref[...], x,
                       preferred_element_type=jnp.float32) * dis


def _kfinal_body(deg_ref, s_ref, b_ref, o_ref):
    deg = deg_ref[0]
    o = s_ref[0] * _dis(deg) + b_ref[...]
    idx = (lax.broadcasted_iota(jnp.int32, (NR, 128), 0) * 128
           + lax.broadcasted_iota(jnp.int32, (NR, 128), 1))
    mask = (idx >= 1) & (idx < N - 1)
    m = jnp.max(jnp.where(mask, o, -jnp.inf))
    ex = jnp.where(mask, jnp.exp(o - m), 0.0)
    o_ref[0] = ex / jnp.sum(ex)


_RB = 1280
_NRB = NP // _RB


def _tc_matmul1t(deg_r, xt, WT):
    return pl.pallas_call(
        _k1t_body,
        grid=(B, _NRB),
        in_specs=[
            pl.BlockSpec((1, 1, _RB), lambda b, i: (b, 0, i)),
            pl.BlockSpec((1, F_IN, _RB), lambda b, i: (b, 0, i)),
            pl.BlockSpec((D, F_IN), lambda b, i: (0, 0)),
        ],
        out_specs=pl.BlockSpec((1, D, _RB), lambda b, i: (b, 0, i)),
        out_shape=jax.ShapeDtypeStruct((B, D, NP), jnp.float32),
    )(deg_r, xt, WT)


def _tc_matmul_midt(deg_r, st, bias_c, WT):
    wo = WT.shape[0]
    return pl.pallas_call(
        _kmidt_body,
        grid=(B, _NRB),
        in_specs=[
            pl.BlockSpec((1, 1, _RB), lambda b, i: (b, 0, i)),
            pl.BlockSpec((1, D, _RB), lambda b, i: (b, 0, i)),
            pl.BlockSpec((D, 1), lambda b, i: (0, 0)),
            pl.BlockSpec((wo, D), lambda b, i: (0, 0)),
        ],
        out_specs=pl.BlockSpec((1, wo, _RB), lambda b, i: (b, 0, i)),
        out_shape=jax.ShapeDtypeStruct((B, wo, NP), jnp.float32),
    )(deg_r, st, bias_c, WT)


def _tc_final(deg_r, s4, b3):
    return pl.pallas_call(
        _kfinal_body,
        grid=(B,),
        in_specs=[
            pl.BlockSpec((1, NR, 128), lambda b: (b, 0, 0)),
            pl.BlockSpec((1, NR, 128), lambda b: (b, 0, 0)),
            pl.BlockSpec((1,), lambda b: (0,)),
        ],
        out_specs=pl.BlockSpec((1, NR, 128), lambda b: (b, 0, 0)),
        out_shape=jax.ShapeDtypeStruct((B, NR, 128), jnp.float32),
    )(deg_r, s4, b3)


def kernel(batch_feat, batch_edges, batch_attr, W1, b1, W2, b2, W22, b22, W3, b3):
    xp = jnp.pad(batch_feat, ((0, 0), (0, NP - N), (0, 0)))
    xt = jnp.swapaxes(xp, 1, 2)
    src = jnp.pad(batch_edges[:, 0, :], ((0, 0), (0, EP - E)))
    dst = jnp.pad(batch_edges[:, 1, :], ((0, 0), (0, EP - E)))
    ew = jnp.pad(batch_attr, ((0, 0), (0, EP - E)))
    pk = jnp.bitwise_or(src, jnp.left_shift(dst, 14))

    deg_r = _deg_kernel(pk, ew)

    g1 = _tc_matmul1t(deg_r, xt, W1.T)
    s1 = _spmm_kernel(g1, pk, ew)
    g2 = _tc_matmul_midt(deg_r, s1, b1.reshape(D, 1), W2.T)
    s2 = _spmm_kernel(g2, pk, ew)
    g3 = _tc_matmul_midt(deg_r, s2, b2.reshape(D, 1), W22.T)
    s3 = _spmm_kernel(g3, pk, ew)

    g4 = _tc_matmul_midt(deg_r, s3, b22.reshape(D, 1), W3.T)
    s4 = _spmv_kernel(g4, pk, ew)
    out = _tc_final(deg_r.reshape(B, NR, 128), s4.reshape(B, NR, 128), b3)
    return out.reshape(B, NP)[:, 1:N - 1]


# speedup vs baseline: 1.1631x; 1.0750x over previous
import functools
import jax
import jax.numpy as jnp
from jax import lax
from jax.experimental import pallas as pl
from jax.experimental.pallas import tpu as pltpu
from jax.experimental.pallas import tpu_sc as plsc

B = 4
N = 10000
NP = 10240
NR = NP // 128
E = 320000
EP = 327680
ER = EP // 128
F_IN = 128
D = 64

NC = 2
NS = 16
EPT = EP // NS
NBLK = EPT // 2048

_mesh = plsc.VectorSubcoreMesh(core_axis_name="c", subcore_axis_name="s")


NRED = NP // 10


def _zero_flat(ref, nwords):
    def body(i, _):
        ref[pl.ds(16 * i, 16)] = jnp.zeros((16,), jnp.float32)
        return 0
    lax.fori_loop(0, nwords // 16, body, 0)


def _slab_reduce_out(acc, slab, tmp, red, sid, out_slice):
    pltpu.sync_copy(acc, slab.at[sid])
    plsc.subcore_barrier()

    @pl.when(sid < 10)
    def _():
        pltpu.sync_copy(slab.at[0, pl.ds(NRED * sid, NRED)], red)

        def addk(k, _):
            pltpu.sync_copy(slab.at[k, pl.ds(NRED * sid, NRED)], tmp)

            def vadd(i, _):
                red[pl.ds(16 * i, 16)] = (red[pl.ds(16 * i, 16)]
                                          + tmp[pl.ds(16 * i, 16)])
                return 0
            lax.fori_loop(0, NRED // 16, vadd, 0)
            return 0
        lax.fori_loop(1, NS, addk, 0)
        out_slice(sid, red)
    plsc.subcore_barrier()


@functools.partial(
    pl.kernel,
    out_type=jax.ShapeDtypeStruct((B // 2, 1, NP), jnp.float32),
    mesh=_mesh,
    compiler_params=pltpu.CompilerParams(needs_layout_passes=False, use_tc_tiling_on_sc=False),
    scratch_types=[
        pltpu.VMEM((EPT,), jnp.int32),
        pltpu.VMEM((EPT,), jnp.float32),
        pltpu.VMEM((NP,), jnp.float32),
        pltpu.VMEM((NRED,), jnp.float32),
        pltpu.VMEM((NRED,), jnp.float32),
        pltpu.VMEM_SHARED((NS, NP), jnp.float32),
    ],
)
def _deg_kernel(pk_hbm, ew_hbm, deg_hbm, pbuf, wbuf, acc, tmp, red, slab):
    cid = lax.axis_index("c")
    sid = lax.axis_index("s")
    e0 = sid * EPT

    def per_graph(gi, _):
        b = cid
        _zero_flat(acc, NP)
        pltpu.sync_copy(pk_hbm.at[b, pl.ds(e0, EPT)], pbuf)
        pltpu.sync_copy(ew_hbm.at[b, pl.ds(e0, EPT)], wbuf)

        @plsc.parallel_loop(0, EPT // 16, step=1, unroll=8)
        def _(j):
            dvec = lax.shift_right_logical(pbuf[pl.ds(16 * j, 16)], 14)
            plsc.addupdate_scatter(acc, [dvec], wbuf[pl.ds(16 * j, 16)])

        def out_slice(t, red_ref):
            pltpu.sync_copy(red_ref, deg_hbm.at[b, 0, pl.ds(NRED * t, NRED)])
        _slab_reduce_out(acc, slab, tmp, red, sid, out_slice)
        return 0
    lax.fori_loop(0, 1, per_graph, 0)


EB = EP // 2048


@functools.partial(
    pl.kernel,
    out_type=jax.ShapeDtypeStruct((B // 2, D, NP), jnp.float32),
    mesh=_mesh,
    compiler_params=pltpu.CompilerParams(needs_layout_passes=False, use_tc_tiling_on_sc=False),
    scratch_types=[
        pltpu.VMEM((2 * 2048,), jnp.int32),
        pltpu.VMEM((2 * 2048,), jnp.float32),
        pltpu.VMEM((NP,), jnp.float32),
        pltpu.VMEM((NP,), jnp.float32),
        pltpu.VMEM((NP,), jnp.float32),
        pltpu.VMEM((NP,), jnp.float32),
        pltpu.VMEM((NP,), jnp.float32),
        pltpu.VMEM((NP,), jnp.float32),
        pltpu.VMEM((NP,), jnp.float32),
        pltpu.VMEM((NP,), jnp.float32),
        pltpu.SemaphoreType.DMA,
        pltpu.SemaphoreType.DMA,
    ],
)
def _spmm_kernel(g_hbm, pk_hbm, ew_hbm, s_hbm,
                 pbuf, wbuf, gc0, gc1, gc2, gc3, ac0, ac1, ac2, ac3,
                 es0, es1):
    cid = lax.axis_index("c")
    sid = lax.axis_index("s")
    gcs = (gc0, gc1, gc2, gc3)
    acs = (ac0, ac1, ac2, ac3)
    c0 = 4 * sid

    def issue(b, k, slot, sem):
        for hbm, buf in ((pk_hbm, pbuf), (ew_hbm, wbuf)):
            pltpu.async_copy(hbm.at[b, pl.ds(2048 * k, 2048)],
                             buf.at[pl.ds(slot * 2048, 2048)], sem)

    def drain(b, slot, sem):
        for hbm, buf in ((pk_hbm, pbuf), (ew_hbm, wbuf)):
            pltpu.make_async_copy(hbm.at[b, pl.ds(0, 2048)],
                                  buf.at[pl.ds(slot * 2048, 2048)], sem).wait()

    def process(slot):
        @plsc.parallel_loop(0, 128, step=1, unroll=8)
        def _(j):
            off = slot * 2048 + 16 * j
            pvec = pbuf[pl.ds(off, 16)]
            svec = lax.bitwise_and(pvec, 16383)
            dvec = lax.shift_right_logical(pvec, 14)
            wvec = wbuf[pl.ds(off, 16)]
            for c in range(4):
                vals = plsc.load_gather(gcs[c], [svec])
                plsc.addupdate_scatter(acs[c], [dvec], vals * wvec)

    def per_graph(gi, _):
        b = cid
        for c in range(4):
            pltpu.sync_copy(g_hbm.at[b, c0 + c], gcs[c])
            _zero_flat(acs[c], NP)
        issue(b, 0, 0, es0)

        def pair(k2, _):
            k = 2 * k2
            issue(b, k + 1, 1, es1)
            drain(b, 0, es0)
            process(0)

            @pl.when(k2 < EB // 2 - 1)
            def _():
                issue(b, k + 2, 0, es0)
            drain(b, 1, es1)
            process(1)
            return 0
        lax.fori_loop(0, EB // 2, pair, 0)

        for c in range(4):
            pltpu.sync_copy(acs[c], s_hbm.at[b, c0 + c])
        return 0
    lax.fori_loop(0, 1, per_graph, 0)


@functools.partial(
    pl.kernel,
    out_type=jax.ShapeDtypeStruct((B // 2, 1, NP), jnp.float32),
    mesh=_mesh,
    compiler_params=pltpu.CompilerParams(needs_layout_passes=False, use_tc_tiling_on_sc=False),
    scratch_types=[
        pltpu.VMEM((EPT,), jnp.int32),
        pltpu.VMEM((EPT,), jnp.float32),
        pltpu.VMEM((NP,), jnp.float32),
        pltpu.VMEM((NP,), jnp.float32),
        pltpu.VMEM((NRED,), jnp.float32),
        pltpu.VMEM((NRED,), jnp.float32),
        pltpu.VMEM_SHARED((NS, NP), jnp.float32),
    ],
)
def _spmv_kernel(g_hbm, pk_hbm, ew_hbm, s_hbm,
                 pbuf, wbuf, gvec, acc, tmp, red, slab):
    cid = lax.axis_index("c")
    sid = lax.axis_index("s")
    e0 = sid * EPT

    def per_graph(gi, _):
        b = cid
        pltpu.sync_copy(g_hbm.at[b, 0], gvec)
        _zero_flat(acc, NP)
        pltpu.sync_copy(pk_hbm.at[b, pl.ds(e0, EPT)], pbuf)
        pltpu.sync_copy(ew_hbm.at[b, pl.ds(e0, EPT)], wbuf)

        @plsc.parallel_loop(0, EPT // 16, step=1, unroll=8)
        def _(j):
            pvec = pbuf[pl.ds(16 * j, 16)]
            svec = lax.bitwise_and(pvec, 16383)
            dvec = lax.shift_right_logical(pvec, 14)
            vals = plsc.load_gather(gvec, [svec])
            plsc.addupdate_scatter(acc, [dvec], vals * wbuf[pl.ds(16 * j, 16)])

        def out_slice(t, red_ref):
            pltpu.sync_copy(red_ref, s_hbm.at[b, 0, pl.ds(NRED * t, NRED)])
        _slab_reduce_out(acc, slab, tmp, red, sid, out_slice)
        return 0
    lax.fori_loop(0, 1, per_graph, 0)


def _dis(deg):
    return jnp.where(deg > 0, lax.rsqrt(jnp.where(deg > 0, deg, 1.0)), 0.0)


def _k1t_body(deg_ref, xt_ref, wt_ref, o_ref):
    dis = _dis(deg_ref[0])
    o_ref[0] = jnp.dot(wt_ref[...], xt_ref[0],
                       preferred_element_type=jnp.float32) * dis


def _kmidt_body(deg_ref, st_ref, b_ref, wt_ref, o_ref):
    dis = _dis(deg_ref[0])
    x = jnp.maximum(st_ref[0] * dis + b_ref[...], 0.0)
    o_ref[0] = jnp.dot(wt_ref[...], x,
                       preferred_element_type=jnp.float32) * dis


def _kfinal_body(deg_ref, s_ref, b_ref, o_ref):
    deg = deg_ref[0]
    o = s_ref[0] * _dis(deg) + b_ref[...]
    idx = (lax.broadcasted_iota(jnp.int32, (NR, 128), 0) * 128
           + lax.broadcasted_iota(jnp.int32, (NR, 128), 1))
    mask = (idx >= 1) & (idx < N - 1)
    m = jnp.max(jnp.where(mask, o, -jnp.inf))
    ex = jnp.where(mask, jnp.exp(o - m), 0.0)
    o_ref[0] = ex / jnp.sum(ex)


_RB = 1280
_NRB = NP // _RB


def _tc_matmul1t(deg_r, xt, WT):
    nb = xt.shape[0]
    return pl.pallas_call(
        _k1t_body,
        grid=(nb, _NRB),
        in_specs=[
            pl.BlockSpec((1, 1, _RB), lambda b, i: (b, 0, i)),
            pl.BlockSpec((1, F_IN, _RB), lambda b, i: (b, 0, i)),
            pl.BlockSpec((D, F_IN), lambda b, i: (0, 0)),
        ],
        out_specs=pl.BlockSpec((1, D, _RB), lambda b, i: (b, 0, i)),
        out_shape=jax.ShapeDtypeStruct((nb, D, NP), jnp.float32),
    )(deg_r, xt, WT)


def _tc_matmul_midt(deg_r, st, bias_c, WT):
    wo = WT.shape[0]
    nb = st.shape[0]
    return pl.pallas_call(
        _kmidt_body,
        grid=(nb, _NRB),
        in_specs=[
            pl.BlockSpec((1, 1, _RB), lambda b, i: (b, 0, i)),
            pl.BlockSpec((1, D, _RB), lambda b, i: (b, 0, i)),
            pl.BlockSpec((D, 1), lambda b, i: (0, 0)),
            pl.BlockSpec((wo, D), lambda b, i: (0, 0)),
        ],
        out_specs=pl.BlockSpec((1, wo, _RB), lambda b, i: (b, 0, i)),
        out_shape=jax.ShapeDtypeStruct((nb, wo, NP), jnp.float32),
    )(deg_r, st, bias_c, WT)


def _tc_final(deg_r, s4, b3):
    nb = s4.shape[0]
    return pl.pallas_call(
        _kfinal_body,
        grid=(nb,),
        in_specs=[
            pl.BlockSpec((1, NR, 128), lambda b: (b, 0, 0)),
            pl.BlockSpec((1, NR, 128), lambda b: (b, 0, 0)),
            pl.BlockSpec((1,), lambda b: (0,)),
        ],
        out_specs=pl.BlockSpec((1, NR, 128), lambda b: (b, 0, 0)),
        out_shape=jax.ShapeDtypeStruct((nb, NR, 128), jnp.float32),
    )(deg_r, s4, b3)


def kernel(batch_feat, batch_edges, batch_attr, W1, b1, W2, b2, W22, b22, W3, b3):
    xp = jnp.pad(batch_feat, ((0, 0), (0, NP - N), (0, 0)))
    xt = jnp.swapaxes(xp, 1, 2)
    src = jnp.pad(batch_edges[:, 0, :], ((0, 0), (0, EP - E)))
    dst = jnp.pad(batch_edges[:, 1, :], ((0, 0), (0, EP - E)))
    ew = jnp.pad(batch_attr, ((0, 0), (0, EP - E)))
    pk = jnp.bitwise_or(src, jnp.left_shift(dst, 14))

    def half(lo, hi):
        pkh, ewh, xth = pk[lo:hi], ew[lo:hi], xt[lo:hi]
        deg_r = _deg_kernel(pkh, ewh)
        g1 = _tc_matmul1t(deg_r, xth, W1.T)
        s1 = _spmm_kernel(g1, pkh, ewh)
        g2 = _tc_matmul_midt(deg_r, s1, b1.reshape(D, 1), W2.T)
        s2 = _spmm_kernel(g2, pkh, ewh)
        g3 = _tc_matmul_midt(deg_r, s2, b2.reshape(D, 1), W22.T)
        s3 = _spmm_kernel(g3, pkh, ewh)
        g4 = _tc_matmul_midt(deg_r, s3, b22.reshape(D, 1), W3.T)
        s4 = _spmv_kernel(g4, pkh, ewh)
        out = _tc_final(deg_r.reshape(2, NR, 128), s4.reshape(2, NR, 128), b3)
        return out

    out_a = half(0, 2)
    out_b = half(2, 4)
    out = jnp.concatenate([out_a, out_b], axis=0)
    return out.reshape(B, NP)[:, 1:N - 1]


# speedup vs baseline: 1.1664x; 1.0029x over previous
import functools
import jax
import jax.numpy as jnp
from jax import lax
from jax.experimental import pallas as pl
from jax.experimental.pallas import tpu as pltpu
from jax.experimental.pallas import tpu_sc as plsc

B = 4
N = 10000
NP = 10240
NR = NP // 128
E = 320000
EP = 327680
ER = EP // 128
F_IN = 128
D = 64

NC = 2
NS = 16
EPT = EP // NS
NBLK = EPT // 2048

_mesh = plsc.VectorSubcoreMesh(core_axis_name="c", subcore_axis_name="s")


NRED = NP // 10


def _zero_flat(ref, nwords):
    def body(i, _):
        ref[pl.ds(16 * i, 16)] = jnp.zeros((16,), jnp.float32)
        return 0
    lax.fori_loop(0, nwords // 16, body, 0)


def _slab_reduce_out(acc, slab, tmp, red, sid, out_slice):
    pltpu.sync_copy(acc, slab.at[sid])
    plsc.subcore_barrier()

    @pl.when(sid < 10)
    def _():
        pltpu.sync_copy(slab.at[0, pl.ds(NRED * sid, NRED)], red)

        def addk(k, _):
            pltpu.sync_copy(slab.at[k, pl.ds(NRED * sid, NRED)], tmp)

            def vadd(i, _):
                red[pl.ds(16 * i, 16)] = (red[pl.ds(16 * i, 16)]
                                          + tmp[pl.ds(16 * i, 16)])
                return 0
            lax.fori_loop(0, NRED // 16, vadd, 0)
            return 0
        lax.fori_loop(1, NS, addk, 0)
        out_slice(sid, red)
    plsc.subcore_barrier()


@functools.partial(
    pl.kernel,
    out_type=jax.ShapeDtypeStruct((B // 2, 1, NP), jnp.float32),
    mesh=_mesh,
    compiler_params=pltpu.CompilerParams(needs_layout_passes=False, use_tc_tiling_on_sc=False),
    scratch_types=[
        pltpu.VMEM((EPT,), jnp.int32),
        pltpu.VMEM((EPT,), jnp.float32),
        pltpu.VMEM((NP,), jnp.float32),
        pltpu.VMEM((NRED,), jnp.float32),
        pltpu.VMEM((NRED,), jnp.float32),
        pltpu.VMEM_SHARED((NS, NP), jnp.float32),
    ],
)
def _deg_kernel(pk_hbm, ew_hbm, deg_hbm, pbuf, wbuf, acc, tmp, red, slab):
    cid = lax.axis_index("c")
    sid = lax.axis_index("s")
    e0 = sid * EPT

    def per_graph(gi, _):
        b = cid
        _zero_flat(acc, NP)
        pltpu.sync_copy(pk_hbm.at[b, pl.ds(e0, EPT)], pbuf)
        pltpu.sync_copy(ew_hbm.at[b, pl.ds(e0, EPT)], wbuf)

        @plsc.parallel_loop(0, EPT // 16, step=1, unroll=8)
        def _(j):
            dvec = lax.shift_right_logical(pbuf[pl.ds(16 * j, 16)], 14)
            plsc.addupdate_scatter(acc, [dvec], wbuf[pl.ds(16 * j, 16)])

        def out_slice(t, red_ref):
            pltpu.sync_copy(red_ref, deg_hbm.at[b, 0, pl.ds(NRED * t, NRED)])
        _slab_reduce_out(acc, slab, tmp, red, sid, out_slice)
        return 0
    lax.fori_loop(0, 1, per_graph, 0)


EB = EP // 4096


@functools.partial(
    pl.kernel,
    out_type=jax.ShapeDtypeStruct((B // 2, D, NP), jnp.float32),
    mesh=_mesh,
    compiler_params=pltpu.CompilerParams(needs_layout_passes=False, use_tc_tiling_on_sc=False),
    scratch_types=[
        pltpu.VMEM((2 * 4096,), jnp.int32),
        pltpu.VMEM((2 * 4096,), jnp.float32),
        pltpu.VMEM((NP,), jnp.float32),
        pltpu.VMEM((NP,), jnp.float32),
        pltpu.VMEM((NP,), jnp.float32),
        pltpu.VMEM((NP,), jnp.float32),
        pltpu.VMEM((NP,), jnp.float32),
        pltpu.VMEM((NP,), jnp.float32),
        pltpu.VMEM((NP,), jnp.float32),
        pltpu.VMEM((NP,), jnp.float32),
        pltpu.SemaphoreType.DMA,
        pltpu.SemaphoreType.DMA,
    ],
)
def _spmm_kernel(g_hbm, pk_hbm, ew_hbm, s_hbm,
                 pbuf, wbuf, gc0, gc1, gc2, gc3, ac0, ac1, ac2, ac3,
                 es0, es1):
    cid = lax.axis_index("c")
    sid = lax.axis_index("s")
    gcs = (gc0, gc1, gc2, gc3)
    acs = (ac0, ac1, ac2, ac3)
    c0 = 4 * sid

    def issue(b, k, slot, sem):
        for hbm, buf in ((pk_hbm, pbuf), (ew_hbm, wbuf)):
            pltpu.async_copy(hbm.at[b, pl.ds(4096 * k, 4096)],
                             buf.at[pl.ds(slot * 4096, 4096)], sem)

    def drain(b, slot, sem):
        for hbm, buf in ((pk_hbm, pbuf), (ew_hbm, wbuf)):
            pltpu.make_async_copy(hbm.at[b, pl.ds(0, 4096)],
                                  buf.at[pl.ds(slot * 4096, 4096)], sem).wait()

    def process(slot):
        @plsc.parallel_loop(0, 256, step=1, unroll=8)
        def _(j):
            off = slot * 4096 + 16 * j
            pvec = pbuf[pl.ds(off, 16)]
            svec = lax.bitwise_and(pvec, 16383)
            dvec = lax.shift_right_logical(pvec, 14)
            wvec = wbuf[pl.ds(off, 16)]
            for c in range(4):
                vals = plsc.load_gather(gcs[c], [svec])
                plsc.addupdate_scatter(acs[c], [dvec], vals * wvec)

    def per_graph(gi, _):
        b = cid
        for c in range(4):
            pltpu.sync_copy(g_hbm.at[b, c0 + c], gcs[c])
            _zero_flat(acs[c], NP)
        issue(b, 0, 0, es0)

        def pair(k2, _):
            k = 2 * k2
            issue(b, k + 1, 1, es1)
            drain(b, 0, es0)
            process(0)

            @pl.when(k2 < EB // 2 - 1)
            def _():
                issue(b, k + 2, 0, es0)
            drain(b, 1, es1)
            process(1)
            return 0
        lax.fori_loop(0, EB // 2, pair, 0)

        for c in range(4):
            pltpu.sync_copy(acs[c], s_hbm.at[b, c0 + c])
        return 0
    lax.fori_loop(0, 1, per_graph, 0)


@functools.partial(
    pl.kernel,
    out_type=jax.ShapeDtypeStruct((B // 2, 1, NP), jnp.float32),
    mesh=_mesh,
    compiler_params=pltpu.CompilerParams(needs_layout_passes=False, use_tc_tiling_on_sc=False),
    scratch_types=[
        pltpu.VMEM((EPT,), jnp.int32),
        pltpu.VMEM((EPT,), jnp.float32),
        pltpu.VMEM((NP,), jnp.float32),
        pltpu.VMEM((NP,), jnp.float32),
        pltpu.VMEM((NRED,), jnp.float32),
        pltpu.VMEM((NRED,), jnp.float32),
        pltpu.VMEM_SHARED((NS, NP), jnp.float32),
    ],
)
def _spmv_kernel(g_hbm, pk_hbm, ew_hbm, s_hbm,
                 pbuf, wbuf, gvec, acc, tmp, red, slab):
    cid = lax.axis_index("c")
    sid = lax.axis_index("s")
    e0 = sid * EPT

    def per_graph(gi, _):
        b = cid
        pltpu.sync_copy(g_hbm.at[b, 0], gvec)
        _zero_flat(acc, NP)
        pltpu.sync_copy(pk_hbm.at[b, pl.ds(e0, EPT)], pbuf)
        pltpu.sync_copy(ew_hbm.at[b, pl.ds(e0, EPT)], wbuf)

        @plsc.parallel_loop(0, EPT // 16, step=1, unroll=8)
        def _(j):
            pvec = pbuf[pl.ds(16 * j, 16)]
            svec = lax.bitwise_and(pvec, 16383)
            dvec = lax.shift_right_logical(pvec, 14)
            vals = plsc.load_gather(gvec, [svec])
            plsc.addupdate_scatter(acc, [dvec], vals * wbuf[pl.ds(16 * j, 16)])

        def out_slice(t, red_ref):
            pltpu.sync_copy(red_ref, s_hbm.at[b, 0, pl.ds(NRED * t, NRED)])
        _slab_reduce_out(acc, slab, tmp, red, sid, out_slice)
        return 0
    lax.fori_loop(0, 1, per_graph, 0)


def _dis(deg):
    return jnp.where(deg > 0, lax.rsqrt(jnp.where(deg > 0, deg, 1.0)), 0.0)


def _k1t_body(deg_ref, xt_ref, wt_ref, o_ref):
    dis = _dis(deg_ref[0])
    o_ref[0] = jnp.dot(wt_ref[...], xt_ref[0],
                       preferred_element_type=jnp.float32) * dis


def _kmidt_body(deg_ref, st_ref, b_ref, wt_ref, o_ref):
    dis = _dis(deg_ref[0])
    x = jnp.maximum(st_ref[0] * dis + b_ref[...], 0.0)
    o_ref[0] = jnp.dot(wt_ref[...], x,
                       preferred_element_type=jnp.float32) * dis


def _kfinal_body(deg_ref, s_ref, b_ref, o_ref):
    deg = deg_ref[0]
    o = s_ref[0] * _dis(deg) + b_ref[...]
    idx = (lax.broadcasted_iota(jnp.int32, (NR, 128), 0) * 128
           + lax.broadcasted_iota(jnp.int32, (NR, 128), 1))
    mask = (idx >= 1) & (idx < N - 1)
    m = jnp.max(jnp.where(mask, o, -jnp.inf))
    ex = jnp.where(mask, jnp.exp(o - m), 0.0)
    o_ref[0] = ex / jnp.sum(ex)


_RB = 1280
_NRB = NP // _RB


def _tc_matmul1t(deg_r, xt, WT):
    nb = xt.shape[0]
    return pl.pallas_call(
        _k1t_body,
        grid=(nb, _NRB),
        in_specs=[
            pl.BlockSpec((1, 1, _RB), lambda b, i: (b, 0, i)),
            pl.BlockSpec((1, F_IN, _RB), lambda b, i: (b, 0, i)),
            pl.BlockSpec((D, F_IN), lambda b, i: (0, 0)),
        ],
        out_specs=pl.BlockSpec((1, D, _RB), lambda b, i: (b, 0, i)),
        out_shape=jax.ShapeDtypeStruct((nb, D, NP), jnp.float32),
    )(deg_r, xt, WT)


def _tc_matmul_midt(deg_r, st, bias_c, WT):
    wo = WT.shape[0]
    nb = st.shape[0]
    return pl.pallas_call(
        _kmidt_body,
        grid=(nb, _NRB),
        in_specs=[
            pl.BlockSpec((1, 1, _RB), lambda b, i: (b, 0, i)),
            pl.BlockSpec((1, D, _RB), lambda b, i: (b, 0, i)),
            pl.BlockSpec((D, 1), lambda b, i: (0, 0)),
            pl.BlockSpec((wo, D), lambda b, i: (0, 0)),
        ],
        out_specs=pl.BlockSpec((1, wo, _RB), lambda b, i: (b, 0, i)),
        out_shape=jax.ShapeDtypeStruct((nb, wo, NP), jnp.float32),
    )(deg_r, st, bias_c, WT)


def _tc_final(deg_r, s4, b3):
    nb = s4.shape[0]
    return pl.pallas_call(
        _kfinal_body,
        grid=(nb,),
        in_specs=[
            pl.BlockSpec((1, NR, 128), lambda b: (b, 0, 0)),
            pl.BlockSpec((1, NR, 128), lambda b: (b, 0, 0)),
            pl.BlockSpec((1,), lambda b: (0,)),
        ],
        out_specs=pl.BlockSpec((1, NR, 128), lambda b: (b, 0, 0)),
        out_shape=jax.ShapeDtypeStruct((nb, NR, 128), jnp.float32),
    )(deg_r, s4, b3)


def kernel(batch_feat, batch_edges, batch_attr, W1, b1, W2, b2, W22, b22, W3, b3):
    xp = jnp.pad(batch_feat, ((0, 0), (0, NP - N), (0, 0)))
    xt = jnp.swapaxes(xp, 1, 2)
    src = jnp.pad(batch_edges[:, 0, :], ((0, 0), (0, EP - E)))
    dst = jnp.pad(batch_edges[:, 1, :], ((0, 0), (0, EP - E)))
    ew = jnp.pad(batch_attr, ((0, 0), (0, EP - E)))
    pk = jnp.bitwise_or(src, jnp.left_shift(dst, 14))

    def half(lo, hi):
        pkh, ewh, xth = pk[lo:hi], ew[lo:hi], xt[lo:hi]
        deg_r = _deg_kernel(pkh, ewh)
        g1 = _tc_matmul1t(deg_r, xth, W1.T)
        s1 = _spmm_kernel(g1, pkh, ewh)
        g2 = _tc_matmul_midt(deg_r, s1, b1.reshape(D, 1), W2.T)
        s2 = _spmm_kernel(g2, pkh, ewh)
        g3 = _tc_matmul_midt(deg_r, s2, b2.reshape(D, 1), W22.T)
        s3 = _spmm_kernel(g3, pkh, ewh)
        g4 = _tc_matmul_midt(deg_r, s3, b22.reshape(D, 1), W3.T)
        s4 = _spmv_kernel(g4, pkh, ewh)
        out = _tc_final(deg_r.reshape(2, NR, 128), s4.reshape(2, NR, 128), b3)
        return out

    out_a = half(0, 2)
    out_b = half(2, 4)
    out = jnp.concatenate([out_a, out_b], axis=0)
    return out.reshape(B, NP)[:, 1:N - 1]
